# baseline (speedup 1.0000x reference)
import jax
import jax.numpy as jnp
from jax import lax
from jax.experimental import pallas as pl
from jax.experimental.pallas import tpu as pltpu
from jax.experimental.pallas import tpu_sc as plsc

N = 4096
T = 128
NT = N // T
NC = 2
NS = 16
NW = NC * NS
STEPS = 17
STRIPW = T + 16
NPACK = N * (N + 1) // 2
ACLAMP = NPACK - STRIPW
TT1 = T + 1


def _body(v_hbm, out_hbm, strip0, strip1, tile0, tile1, tt0, tt1, tprobe,
          sem_in0, sem_in1, sem_out0, sem_out1):
    wid = lax.axis_index("s") * NC + lax.axis_index("c")
    iota16 = lax.iota(jnp.int32, 16)
    strips = [strip0, strip1]
    tiles = [tile0, tile1]
    tts = [tt0, tt1]
    sem_ins = [sem_in0, sem_in1]
    sem_outs = [sem_out0, sem_out1]

    def decode(mm):
        first = wid < (NT - mm)
        ti = jnp.where(first, wid, wid - NT + mm)
        tj = jnp.where(first, wid + mm, wid)
        valid = jnp.logical_or(mm < 16, wid < 16)
        return ti, tj, valid

    def row_start(i, j0):
        return i * N - (i * (i - 1)) // 2 - i + j0

    def fire(mm, par):
        ti, tj, _ = decode(mm)
        i0 = ti * T
        j0 = tj * T
        strip = strips[par]
        sem = sem_ins[par]

        s0 = row_start(i0, j0)
        d0 = N - 1 - i0

        with jax.named_scope("fire"):
            @plsc.parallel_loop(0, T, unroll=8)
            def go(k):
                s = s0 + k * d0 - ((k * (k - 1)) >> 1)
                a = jnp.minimum((s >> 4) << 4, ACLAMP)
                a = pl.multiple_of(a, 16)
                pltpu.async_copy(
                    v_hbm.at[pl.ds(a, STRIPW)],
                    strip.at[pl.ds(k * STRIPW, STRIPW)],
                    sem,
                )

    def wait_tilebytes(par, buf):
        pltpu.make_async_copy(
            out_hbm.at[pl.ds(0, T), pl.ds(0, T)], buf, sem_outs[par]
        ).wait()

    def step(m, par):
        pi, pj, pvalid = decode(m - 2)

        @pl.when(jnp.logical_and(m >= 2, pvalid))
        def _():
          with jax.named_scope("drain_out"):
            wait_tilebytes(par, tiles[par])

            @pl.when(pi != pj)
            def _():
                wait_tilebytes(par, tiles[par])

        ni, nj, nvalid = decode(m + 1)

        @pl.when(jnp.logical_and(m + 1 < STEPS, nvalid))
        def _():
            fire(m + 1, 1 - par)

        ti, tj, valid = decode(m)

        @pl.when(valid)
        def _():
            i0 = ti * T
            j0 = tj * T
            strip = strips[par]
            tile = tiles[par]
            tile_t = tts[par]
            with jax.named_scope("drain_in"):
                pltpu.make_async_copy(
                    v_hbm.at[pl.ds(0, T * STRIPW)], strip, sem_ins[par]
                ).wait()

            s0 = row_start(i0, j0)
            d0 = N - 1 - i0

            with jax.named_scope("rowfix"):
                @plsc.parallel_loop(0, T, unroll=2)
                def rowfix(k):
                    s = s0 + k * d0 - ((k * (k - 1)) >> 1)
                    a = jnp.minimum((s >> 4) << 4, ACLAMP)
                    r = s - a
                    krow = jnp.full((16,), k, jnp.int32)
                    base = k * STRIPW + r
                    for g in range(T // 16):
                        cols = g * 16 + iota16
                        vals = plsc.load_gather(strip, [base + cols])
                        tile[k, pl.ds(g * 16, 16)] = vals
                        plsc.store_scatter(tprobe, [cols * TT1 + krow], vals)

            diag = ti == tj

            @pl.when(diag)
            def _():
                @plsc.parallel_loop(0, T, unroll=2)
                def merge(k):
                    for g in range(T // 16):
                        cols = g * 16 + iota16
                        a = tile[k, pl.ds(g * 16, 16)]
                        bt = plsc.load_gather(tprobe, [k * TT1 + cols])
                        tile[k, pl.ds(g * 16, 16)] = jnp.where(
                            cols >= k, a, bt
                        )

            @pl.when(jnp.logical_not(diag))
            def _():
                @plsc.parallel_loop(0, T, unroll=2)
                def densify(c):
                    for g in range(T // 16):
                        vals = plsc.load_gather(
                            tprobe, [c * TT1 + g * 16 + iota16]
                        )
                        tile_t[c, pl.ds(g * 16, 16)] = vals

            i0a = pl.multiple_of(i0, T)
            j0a = pl.multiple_of(j0, T)
            pltpu.async_copy(
                tile, out_hbm.at[pl.ds(i0a, T), pl.ds(j0a, T)], sem_outs[par]
            )

            @pl.when(jnp.logical_not(diag))
            def _():
                pltpu.async_copy(
                    tile_t,
                    out_hbm.at[pl.ds(j0a, T), pl.ds(i0a, T)],
                    sem_outs[par],
                )

    def two_steps(q, carry):
        step(2 * q, 0)

        @pl.when(2 * q + 1 < STEPS)
        def _():
            step(2 * q + 1, 1)

        return carry

    fire(0, 0)
    lax.fori_loop(0, (STEPS + 1) // 2, two_steps, 0)

    for mm in (STEPS - 2, STEPS - 1):
        fi, fj, fvalid = decode(mm)

        @pl.when(fvalid)
        def _():
            wait_tilebytes(mm & 1, tiles[mm & 1])

            @pl.when(fi != fj)
            def _():
                wait_tilebytes(mm & 1, tiles[mm & 1])


@jax.jit
def kernel(upper_tri_vector):
    mesh = plsc.VectorSubcoreMesh(
        core_axis_name="c", subcore_axis_name="s", num_cores=NC
    )
    fn = pl.kernel(
        _body,
        out_type=jax.ShapeDtypeStruct((N, N), jnp.float32),
        mesh=mesh,
        scratch_types=[
            pltpu.VMEM((T * STRIPW,), jnp.float32),
            pltpu.VMEM((T * STRIPW,), jnp.float32),
            pltpu.VMEM((T, T), jnp.float32),
            pltpu.VMEM((T, T), jnp.float32),
            pltpu.VMEM((T, T), jnp.float32),
            pltpu.VMEM((T, T), jnp.float32),
            pltpu.VMEM((T * TT1,), jnp.float32),
            pltpu.SemaphoreType.DMA,
            pltpu.SemaphoreType.DMA,
            pltpu.SemaphoreType.DMA,
            pltpu.SemaphoreType.DMA,
        ],
        compiler_params=pltpu.CompilerParams(needs_layout_passes=False),
    )
    return fn(upper_tri_vector)


# speedup vs baseline: 1.0259x; 1.0259x over previous
import jax
import jax.numpy as jnp
from jax import lax
from jax.experimental import pallas as pl
from jax.experimental.pallas import tpu as pltpu
from jax.experimental.pallas import tpu_sc as plsc

N = 4096
T = 128
NT = N // T
NC = 2
NS = 16
NW = NC * NS
STEPS = 17
STRIPW = T + 16
NPACK = N * (N + 1) // 2
ACLAMP = NPACK - STRIPW
TT1 = T + 1


def _body(v_hbm, out_hbm, strip0, strip1, tile0, tile1, tt0, tt1, tprobe,
          sem_in0, sem_in1, sem_out0, sem_out1):
    wid = lax.axis_index("s") * NC + lax.axis_index("c")
    iota16 = lax.iota(jnp.int32, 16)
    strips = [strip0, strip1]
    tiles = [tile0, tile1]
    tts = [tt0, tt1]
    sem_ins = [sem_in0, sem_in1]
    sem_outs = [sem_out0, sem_out1]

    def decode(mm):
        first = wid < (NT - mm)
        ti = jnp.where(first, wid, wid - NT + mm)
        tj = jnp.where(first, wid + mm, wid)
        valid = jnp.logical_or(mm < 16, wid < 16)
        return ti, tj, valid

    def row_start(i, j0):
        return i * N - (i * (i - 1)) // 2 - i + j0

    def fire(mm, par):
        ti, tj, _ = decode(mm)
        i0 = ti * T
        j0 = tj * T
        strip = strips[par]
        sem = sem_ins[par]

        s0 = row_start(i0, j0)
        d0 = N - 1 - i0

        with jax.named_scope("fire"):
            @plsc.parallel_loop(0, T, unroll=4)
            def go(k):
                s = s0 + k * d0 - ((k * (k - 1)) >> 1)
                a = jnp.minimum((s >> 4) << 4, ACLAMP)
                a = pl.multiple_of(a, 16)
                pltpu.async_copy(
                    v_hbm.at[pl.ds(a, STRIPW)],
                    strip.at[pl.ds(k * STRIPW, STRIPW)],
                    sem,
                )

    def wait_tilebytes(par, buf):
        pltpu.make_async_copy(
            out_hbm.at[pl.ds(0, T), pl.ds(0, T)], buf, sem_outs[par]
        ).wait()

    def step(m, par):
        pi, pj, pvalid = decode(m - 2)

        @pl.when(jnp.logical_and(m >= 2, pvalid))
        def _():
          with jax.named_scope("drain_out"):
            wait_tilebytes(par, tiles[par])

            @pl.when(pi != pj)
            def _():
                wait_tilebytes(par, tiles[par])

        ni, nj, nvalid = decode(m + 1)

        @pl.when(jnp.logical_and(m + 1 < STEPS, nvalid))
        def _():
            fire(m + 1, 1 - par)

        ti, tj, valid = decode(m)

        @pl.when(valid)
        def _():
            i0 = ti * T
            j0 = tj * T
            strip = strips[par]
            tile = tiles[par]
            tile_t = tts[par]
            with jax.named_scope("drain_in"):
                pltpu.make_async_copy(
                    v_hbm.at[pl.ds(0, T * STRIPW)], strip, sem_ins[par]
                ).wait()

            s0 = row_start(i0, j0)
            d0 = N - 1 - i0

            with jax.named_scope("rowfix"):
                @plsc.parallel_loop(0, T, unroll=2)
                def rowfix(k):
                    s = s0 + k * d0 - ((k * (k - 1)) >> 1)
                    a = jnp.minimum((s >> 4) << 4, ACLAMP)
                    r = s - a
                    krow = jnp.full((16,), k, jnp.int32)
                    base = k * STRIPW + r
                    for g in range(T // 16):
                        cols = g * 16 + iota16
                        vals = plsc.load_gather(strip, [base + cols])
                        tile[k, pl.ds(g * 16, 16)] = vals
                        plsc.store_scatter(tprobe, [cols * TT1 + krow], vals)

            diag = ti == tj

            @pl.when(diag)
            def _():
                @plsc.parallel_loop(0, T, unroll=2)
                def merge(k):
                    for g in range(T // 16):
                        cols = g * 16 + iota16
                        a = tile[k, pl.ds(g * 16, 16)]
                        bt = plsc.load_gather(tprobe, [k * TT1 + cols])
                        tile[k, pl.ds(g * 16, 16)] = jnp.where(
                            cols >= k, a, bt
                        )

            @pl.when(jnp.logical_not(diag))
            def _():
                @plsc.parallel_loop(0, T, unroll=2)
                def densify(c):
                    for g in range(T // 16):
                        vals = plsc.load_gather(
                            tprobe, [c * TT1 + g * 16 + iota16]
                        )
                        tile_t[c, pl.ds(g * 16, 16)] = vals

            i0a = pl.multiple_of(i0, T)
            j0a = pl.multiple_of(j0, T)
            pltpu.async_copy(
                tile, out_hbm.at[pl.ds(i0a, T), pl.ds(j0a, T)], sem_outs[par]
            )

            @pl.when(jnp.logical_not(diag))
            def _():
                pltpu.async_copy(
                    tile_t,
                    out_hbm.at[pl.ds(j0a, T), pl.ds(i0a, T)],
                    sem_outs[par],
                )

    def two_steps(q, carry):
        step(2 * q, 0)

        @pl.when(2 * q + 1 < STEPS)
        def _():
            step(2 * q + 1, 1)

        return carry

    fire(0, 0)
    lax.fori_loop(0, (STEPS + 1) // 2, two_steps, 0)

    for mm in (STEPS - 2, STEPS - 1):
        fi, fj, fvalid = decode(mm)

        @pl.when(fvalid)
        def _():
            wait_tilebytes(mm & 1, tiles[mm & 1])

            @pl.when(fi != fj)
            def _():
                wait_tilebytes(mm & 1, tiles[mm & 1])


@jax.jit
def kernel(upper_tri_vector):
    mesh = plsc.VectorSubcoreMesh(
        core_axis_name="c", subcore_axis_name="s", num_cores=NC
    )
    fn = pl.kernel(
        _body,
        out_type=jax.ShapeDtypeStruct((N, N), jnp.float32),
        mesh=mesh,
        scratch_types=[
            pltpu.VMEM((T * STRIPW,), jnp.float32),
            pltpu.VMEM((T * STRIPW,), jnp.float32),
            pltpu.VMEM((T, T), jnp.float32),
            pltpu.VMEM((T, T), jnp.float32),
            pltpu.VMEM((T, T), jnp.float32),
            pltpu.VMEM((T, T), jnp.float32),
            pltpu.VMEM((T * TT1,), jnp.float32),
            pltpu.SemaphoreType.DMA,
            pltpu.SemaphoreType.DMA,
            pltpu.SemaphoreType.DMA,
            pltpu.SemaphoreType.DMA,
        ],
        compiler_params=pltpu.CompilerParams(needs_layout_passes=False),
    )
    return fn(upper_tri_vector)
